# baseline (speedup 1.0000x reference)
import jax
import jax.numpy as jnp
from jax import lax
from jax.experimental import pallas as pl
from jax.experimental.pallas import tpu as pltpu
from jax.experimental.pallas import tpu_sc as plsc

RHO = 1.0
BETA = 0.5
TAU = 0.1
EPS = 1e-9
LN2 = 0.6931471805599453

L = 16
LOG2E = 1.4426950408889634


def _bf16r(v):
    b = lax.bitcast_convert_type(v, jnp.int32)
    lsb = jnp.bitwise_and(lax.shift_right_logical(b, 16), 1)
    b = jnp.bitwise_and(b + 0x7FFF + lsb, jnp.int32(-65536))
    return lax.bitcast_convert_type(b, jnp.float32)


def _bf16h(v):
    b = lax.bitcast_convert_type(v, jnp.int32)
    b = jnp.bitwise_and(b + 0x8000, jnp.int32(-65536))
    return lax.bitcast_convert_type(b, jnp.float32)


def _rcp16(x):
    r = 1.0 / x
    return r * (2.0 - x * r)


def _exp16(x):
    x = jnp.maximum(x, -87.0)
    z = x * LOG2E
    zi = (z - 0.5).astype(jnp.int32)
    zf = z - zi.astype(jnp.float32)
    t = zf * LN2
    p = 1.0 + t * (1.0 + t * (1.0 / 2 + t * (1.0 / 6 + t * (1.0 / 24 + t * (
        1.0 / 120 + t * (1.0 / 720 + t * (1.0 / 5040)))))))
    scale = lax.bitcast_convert_type(
        lax.shift_left(zi + 127, jnp.int32(23)), jnp.float32)
    return p * scale


def _log16(x):
    bits = lax.bitcast_convert_type(x, jnp.int32)
    e = jnp.bitwise_and(lax.shift_right_logical(bits, 23), 0xFF)
    ef = (e - 127).astype(jnp.float32)
    mbits = jnp.bitwise_or(jnp.bitwise_and(bits, 0x007FFFFF), 0x3F800000)
    m = lax.bitcast_convert_type(mbits, jnp.float32)
    s = (m - 1.0) * _rcp16(m + 1.0)
    s2 = s * s
    p = 2.0 * s * (1.0 + s2 * (1.0 / 3.0 + s2 * (0.2 + s2 * (1.0 / 7.0 + s2 * (1.0 / 9.0)))))
    return ef * LN2 + p


def _make_sc_kernel(n, k, d, nmem):
    info = plsc.get_sparse_core_info()
    nc, ns = info.num_cores, info.num_subcores
    nw = nc * ns
    tpt = n // nw
    nch = d // L
    mesh = plsc.VectorSubcoreMesh(core_axis_name="c", subcore_axis_name="s")

    def body(y_hbm, p_hbm, lam_hbm, m_hbm, kset_hbm,
             pnew_hbm, lamnew_hbm,
             kset_v, p_v, pnew_v, t_v, y_v, lam_v,
             yp_v, xi_v, lamnew_v,
             gsem0, gsem1, iosem0, iosem1, osem0, osem1):
        wid = lax.axis_index("s") * nc + lax.axis_index("c")
        base = wid * tpt
        gsems = (gsem0, gsem1)
        iosems = (iosem0, iosem1)
        osems = (osem0, osem1)

        pltpu.sync_copy(kset_hbm.at[pl.ds(base, tpt)], kset_v)
        pltpu.sync_copy(p_hbm.at[pl.ds(base, tpt)], p_v)

        def fire_in(tl, b):
            tok = base + tl
            pltpu.async_copy(m_hbm.at[kset_v.at[tl]], t_v.at[b], gsems[b])
            pltpu.async_copy(y_hbm.at[tok], y_v.at[b], iosems[b])
            pltpu.async_copy(lam_hbm.at[tok], lam_v.at[b], iosems[b])

        def wait_in(tl, b):
            tok = base + tl
            pltpu.make_async_copy(m_hbm.at[kset_v.at[tl]], t_v.at[b], gsems[b]).wait()
            pltpu.make_async_copy(y_hbm.at[tok], y_v.at[b], iosems[b]).wait()
            pltpu.make_async_copy(lam_hbm.at[tok], lam_v.at[b], iosems[b]).wait()

        def weighted_colsum(b, w0, w1):
            def colsum(c):
                sl = pl.ds(c * L, L)
                acc = jnp.zeros((L,), jnp.float32)
                for j in range(k):
                    w = w0[j] if j < L else w1[j - L]
                    acc = acc + w * _bf16h(t_v[b, j, sl])
                return acc, sl
            return colsum

        def token_compute(t, b):
            p0 = p_v[t, 0:L]
            p1 = p_v[t, L:2 * L]
            psum = jnp.sum(p0) + jnp.sum(p1) + EPS
            inv = _rcp16(jnp.broadcast_to(psum, (L,)))
            pn0 = p0 * inv
            pn1 = p1 * inv

            colsum_pn = weighted_colsum(b, _bf16r(pn0), _bf16r(pn1))

            @plsc.parallel_loop(0, nch, unroll=2)
            def pass1(c):
                acc, sl = colsum_pn(c)
                yp_v[sl] = acc
                xi_v[sl] = _bf16r(lam_v[b, sl] + RHO * (y_v[b, sl] - acc))

            lanes = lax.iota(jnp.int32, L)
            sc0 = sc1 = vv0 = vv1 = q0 = q1 = jnp.zeros((L,), jnp.float32)
            for jb in range(k // 4):
                def chunk(c, accs, jb=jb):
                    sl = pl.ds(c * L, L)
                    xi = xi_v[sl]
                    yp = yp_v[sl]
                    out = []
                    for r in range(4):
                        tv = t_v[b, jb * 4 + r, sl]
                        out.append(accs[3 * r] + _bf16h(tv) * xi)
                        out.append(accs[3 * r + 1] + tv * yp)
                        out.append(accs[3 * r + 2] + tv * tv)
                    return tuple(out)
                accs = plsc.parallel_loop(
                    0, nch,
                    carry=tuple(jnp.zeros((L,), jnp.float32) for _ in range(12)),
                )(chunk)
                for r in range(4):
                    j = jb * 4 + r
                    msk = lanes == (j % L)
                    ssc = jnp.sum(accs[3 * r])
                    svv = jnp.sum(accs[3 * r + 1])
                    sqq = jnp.sum(accs[3 * r + 2])
                    if j < L:
                        sc0 = jnp.where(msk, ssc, sc0)
                        vv0 = jnp.where(msk, svv, vv0)
                        q0 = jnp.where(msk, sqq, q0)
                    else:
                        sc1 = jnp.where(msk, ssc, sc1)
                        vv1 = jnp.where(msk, svv, vv1)
                        q1 = jnp.where(msk, sqq, q1)

            sm0 = q0 - 2.0 * vv0
            sm1 = q1 - 2.0 * vv1
            lg0 = _log16(pn0 + EPS) - BETA * sc0 - TAU * sm0
            lg1 = _log16(pn1 + EPS) - BETA * sc1 - TAU * sm1
            mx = jnp.maximum(jnp.max(lg0), jnp.max(lg1))
            e0 = _exp16(lg0 - mx)
            e1 = _exp16(lg1 - mx)
            iz = _rcp16(jnp.broadcast_to(jnp.sum(e0) + jnp.sum(e1), (L,)))
            pw0 = e0 * iz
            pw1 = e1 * iz
            pnew_v[t, 0:L] = pw0
            pnew_v[t, L:2 * L] = pw1

            colsum_pw = weighted_colsum(b, _bf16r(pw0), _bf16r(pw1))

            @plsc.parallel_loop(0, nch, unroll=2)
            def pass3(c):
                acc, sl = colsum_pw(c)
                lamnew_v[b, sl] = lam_v[b, sl] + RHO * (y_v[b, sl] - acc)

        fire_in(0, 0)

        def pair_body(i, _):
            for tt in range(2):
                b = tt
                tl = 2 * i + tt

                @pl.when(tl + 1 < tpt)
                def _():
                    fire_in(tl + 1, 1 - b)
                wait_in(tl, b)

                @pl.when(tl >= 2)
                def _():
                    pltpu.make_async_copy(
                        lamnew_v.at[b], lamnew_hbm.at[base + tl - 2],
                        osems[b]).wait()
                token_compute(tl, b)
                pltpu.async_copy(lamnew_v.at[b], lamnew_hbm.at[base + tl],
                                 osems[b])
            return 0

        lax.fori_loop(0, tpt // 2, pair_body, 0, unroll=False)
        pltpu.make_async_copy(lamnew_v.at[0], lamnew_hbm.at[base + tpt - 2],
                              osems[0]).wait()
        pltpu.make_async_copy(lamnew_v.at[1], lamnew_hbm.at[base + tpt - 1],
                              osems[1]).wait()
        pltpu.sync_copy(pnew_v, pnew_hbm.at[pl.ds(base, tpt)])

    return pl.kernel(
        body,
        out_type=(jax.ShapeDtypeStruct((n, k), jnp.float32),
                  jax.ShapeDtypeStruct((n, d), jnp.float32)),
        mesh=mesh,
        compiler_params=pltpu.CompilerParams(needs_layout_passes=False),
        scratch_types=[
            pltpu.VMEM((tpt, k), jnp.int32),
            pltpu.VMEM((tpt, k), jnp.float32),
            pltpu.VMEM((tpt, k), jnp.float32),
            pltpu.VMEM((2, k, d), jnp.float32),
            pltpu.VMEM((2, d), jnp.float32),
            pltpu.VMEM((2, d), jnp.float32),
            pltpu.VMEM((d,), jnp.float32),
            pltpu.VMEM((d,), jnp.float32),
            pltpu.VMEM((2, d), jnp.float32),
            pltpu.SemaphoreType.DMA,
            pltpu.SemaphoreType.DMA,
            pltpu.SemaphoreType.DMA,
            pltpu.SemaphoreType.DMA,
            pltpu.SemaphoreType.DMA,
            pltpu.SemaphoreType.DMA,
        ],
    )


def kernel(Y, P, Lam, M, Kset):
    n, d = Y.shape
    k = P.shape[1]
    nmem = M.shape[0]
    sc = _make_sc_kernel(n, k, d, nmem)
    return sc(Y, P, Lam, M, Kset)


# speedup vs baseline: 1.1010x; 1.1010x over previous
import jax
import jax.numpy as jnp
from jax import lax
from jax.experimental import pallas as pl
from jax.experimental.pallas import tpu as pltpu
from jax.experimental.pallas import tpu_sc as plsc

RHO = 1.0
BETA = 0.5
TAU = 0.1
EPS = 1e-9
LN2 = 0.6931471805599453

L = 16
LOG2E = 1.4426950408889634


def _bf16r(v):
    b = lax.bitcast_convert_type(v, jnp.int32)
    lsb = jnp.bitwise_and(lax.shift_right_logical(b, 16), 1)
    b = jnp.bitwise_and(b + 0x7FFF + lsb, jnp.int32(-65536))
    return lax.bitcast_convert_type(b, jnp.float32)


def _bf16h(v):
    b = lax.bitcast_convert_type(v, jnp.int32)
    b = jnp.bitwise_and(b + 0x8000, jnp.int32(-65536))
    return lax.bitcast_convert_type(b, jnp.float32)


def _rcp16(x):
    r = 1.0 / x
    return r * (2.0 - x * r)


def _exp16(x):
    x = jnp.maximum(x, -87.0)
    z = x * LOG2E
    zi = (z - 0.5).astype(jnp.int32)
    zf = z - zi.astype(jnp.float32)
    t = zf * LN2
    p = 1.0 + t * (1.0 + t * (1.0 / 2 + t * (1.0 / 6 + t * (1.0 / 24 + t * (
        1.0 / 120 + t * (1.0 / 720 + t * (1.0 / 5040)))))))
    scale = lax.bitcast_convert_type(
        lax.shift_left(zi + 127, jnp.int32(23)), jnp.float32)
    return p * scale


def _log16(x):
    bits = lax.bitcast_convert_type(x, jnp.int32)
    e = jnp.bitwise_and(lax.shift_right_logical(bits, 23), 0xFF)
    ef = (e - 127).astype(jnp.float32)
    mbits = jnp.bitwise_or(jnp.bitwise_and(bits, 0x007FFFFF), 0x3F800000)
    m = lax.bitcast_convert_type(mbits, jnp.float32)
    s = (m - 1.0) * _rcp16(m + 1.0)
    s2 = s * s
    p = 2.0 * s * (1.0 + s2 * (1.0 / 3.0 + s2 * (0.2 + s2 * (1.0 / 7.0 + s2 * (1.0 / 9.0)))))
    return ef * LN2 + p


def _make_sc_kernel(n, k, d, nmem):
    info = plsc.get_sparse_core_info()
    nc, ns = info.num_cores, info.num_subcores
    nw = nc * ns
    tpt = n // nw
    nch = d // L
    mesh = plsc.VectorSubcoreMesh(core_axis_name="c", subcore_axis_name="s")

    def body(y_hbm, p_hbm, lam_hbm, m_hbm, kset_hbm,
             pnew_hbm, lamnew_hbm,
             kset_v, p_v, pnew_v, t_v, y_v, lam_v,
             yp_v, xi_v, lamnew_v,
             gsem0, gsem1, iosem0, iosem1, osem0, osem1):
        wid = lax.axis_index("s") * nc + lax.axis_index("c")
        base = wid * tpt
        gsems = (gsem0, gsem1)
        iosems = (iosem0, iosem1)
        osems = (osem0, osem1)

        pltpu.sync_copy(kset_hbm.at[pl.ds(base, tpt)], kset_v)
        pltpu.sync_copy(p_hbm.at[pl.ds(base, tpt)], p_v)

        def fire_in(tl, b):
            tok = base + tl
            pltpu.async_copy(m_hbm.at[kset_v.at[tl]], t_v.at[b], gsems[b])
            pltpu.async_copy(y_hbm.at[tok], y_v.at[b], iosems[b])
            pltpu.async_copy(lam_hbm.at[tok], lam_v.at[b], iosems[b])

        def wait_in(tl, b):
            tok = base + tl
            pltpu.make_async_copy(m_hbm.at[kset_v.at[tl]], t_v.at[b], gsems[b]).wait()
            pltpu.make_async_copy(y_hbm.at[tok], y_v.at[b], iosems[b]).wait()
            pltpu.make_async_copy(lam_hbm.at[tok], lam_v.at[b], iosems[b]).wait()

        def weighted_colsum(b, w0, w1, finish):
            for g in range(k // 8):
                ws = [jnp.broadcast_to(w0[g * 8 + r] if g * 8 + r < L
                                       else w1[g * 8 + r - L], (L,))
                      for r in range(8)]

                @plsc.parallel_loop(0, nch)
                def _grp(c, g=g, ws=ws):
                    sl = pl.ds(c * L, L)
                    acc = jnp.zeros((L,), jnp.float32) if g == 0 else yp_v[sl]
                    for r in range(8):
                        acc = acc + ws[r] * _bf16h(t_v[b, g * 8 + r, sl])
                    if g < k // 8 - 1:
                        yp_v[sl] = acc
                    else:
                        finish(sl, acc)

        def token_compute(t, b):
            p0 = p_v[t, 0:L]
            p1 = p_v[t, L:2 * L]
            psum = jnp.sum(p0) + jnp.sum(p1) + EPS
            inv = _rcp16(jnp.broadcast_to(psum, (L,)))
            pn0 = p0 * inv
            pn1 = p1 * inv

            def fin1(sl, acc):
                yp_v[sl] = acc
                xi_v[sl] = _bf16r(lam_v[b, sl] + RHO * (y_v[b, sl] - acc))
            weighted_colsum(b, _bf16r(pn0), _bf16r(pn1), fin1)

            lanes = lax.iota(jnp.int32, L)
            sc0 = sc1 = vv0 = vv1 = q0 = q1 = jnp.zeros((L,), jnp.float32)
            for jb in range(k // 4):
                def chunk(c, accs, jb=jb):
                    sl = pl.ds(c * L, L)
                    xi = xi_v[sl]
                    yp = yp_v[sl]
                    out = []
                    for r in range(4):
                        tv = t_v[b, jb * 4 + r, sl]
                        out.append(accs[3 * r] + _bf16h(tv) * xi)
                        out.append(accs[3 * r + 1] + tv * yp)
                        out.append(accs[3 * r + 2] + tv * tv)
                    return tuple(out)
                accs = plsc.parallel_loop(
                    0, nch,
                    carry=tuple(jnp.zeros((L,), jnp.float32) for _ in range(12)),
                )(chunk)
                for r in range(4):
                    j = jb * 4 + r
                    msk = lanes == (j % L)
                    ssc = jnp.sum(accs[3 * r])
                    svv = jnp.sum(accs[3 * r + 1])
                    sqq = jnp.sum(accs[3 * r + 2])
                    if j < L:
                        sc0 = jnp.where(msk, ssc, sc0)
                        vv0 = jnp.where(msk, svv, vv0)
                        q0 = jnp.where(msk, sqq, q0)
                    else:
                        sc1 = jnp.where(msk, ssc, sc1)
                        vv1 = jnp.where(msk, svv, vv1)
                        q1 = jnp.where(msk, sqq, q1)

            sm0 = q0 - 2.0 * vv0
            sm1 = q1 - 2.0 * vv1
            lg0 = _log16(pn0 + EPS) - BETA * sc0 - TAU * sm0
            lg1 = _log16(pn1 + EPS) - BETA * sc1 - TAU * sm1
            mx = jnp.maximum(jnp.max(lg0), jnp.max(lg1))
            e0 = _exp16(lg0 - mx)
            e1 = _exp16(lg1 - mx)
            iz = _rcp16(jnp.broadcast_to(jnp.sum(e0) + jnp.sum(e1), (L,)))
            pw0 = e0 * iz
            pw1 = e1 * iz
            pnew_v[t, 0:L] = pw0
            pnew_v[t, L:2 * L] = pw1

            def fin3(sl, acc):
                lamnew_v[b, sl] = lam_v[b, sl] + RHO * (y_v[b, sl] - acc)
            weighted_colsum(b, _bf16r(pw0), _bf16r(pw1), fin3)

        fire_in(0, 0)

        def pair_body(i, _):
            for tt in range(2):
                b = tt
                tl = 2 * i + tt

                @pl.when(tl + 1 < tpt)
                def _():
                    fire_in(tl + 1, 1 - b)
                wait_in(tl, b)

                @pl.when(tl >= 2)
                def _():
                    pltpu.make_async_copy(
                        lamnew_v.at[b], lamnew_hbm.at[base + tl - 2],
                        osems[b]).wait()
                token_compute(tl, b)
                pltpu.async_copy(lamnew_v.at[b], lamnew_hbm.at[base + tl],
                                 osems[b])
            return 0

        lax.fori_loop(0, tpt // 2, pair_body, 0, unroll=False)
        pltpu.make_async_copy(lamnew_v.at[0], lamnew_hbm.at[base + tpt - 2],
                              osems[0]).wait()
        pltpu.make_async_copy(lamnew_v.at[1], lamnew_hbm.at[base + tpt - 1],
                              osems[1]).wait()
        pltpu.sync_copy(pnew_v, pnew_hbm.at[pl.ds(base, tpt)])

    return pl.kernel(
        body,
        out_type=(jax.ShapeDtypeStruct((n, k), jnp.float32),
                  jax.ShapeDtypeStruct((n, d), jnp.float32)),
        mesh=mesh,
        compiler_params=pltpu.CompilerParams(needs_layout_passes=False),
        scratch_types=[
            pltpu.VMEM((tpt, k), jnp.int32),
            pltpu.VMEM((tpt, k), jnp.float32),
            pltpu.VMEM((tpt, k), jnp.float32),
            pltpu.VMEM((2, k, d), jnp.float32),
            pltpu.VMEM((2, d), jnp.float32),
            pltpu.VMEM((2, d), jnp.float32),
            pltpu.VMEM((d,), jnp.float32),
            pltpu.VMEM((d,), jnp.float32),
            pltpu.VMEM((2, d), jnp.float32),
            pltpu.SemaphoreType.DMA,
            pltpu.SemaphoreType.DMA,
            pltpu.SemaphoreType.DMA,
            pltpu.SemaphoreType.DMA,
            pltpu.SemaphoreType.DMA,
            pltpu.SemaphoreType.DMA,
        ],
    )


def kernel(Y, P, Lam, M, Kset):
    n, d = Y.shape
    k = P.shape[1]
    nmem = M.shape[0]
    sc = _make_sc_kernel(n, k, d, nmem)
    return sc(Y, P, Lam, M, Kset)


# speedup vs baseline: 1.1480x; 1.0427x over previous
import jax
import jax.numpy as jnp
from jax import lax
from jax.experimental import pallas as pl
from jax.experimental.pallas import tpu as pltpu
from jax.experimental.pallas import tpu_sc as plsc

RHO = 1.0
BETA = 0.5
TAU = 0.1
EPS = 1e-9
LN2 = 0.6931471805599453

L = 16
LOG2E = 1.4426950408889634


def _bf16r(v):
    b = lax.bitcast_convert_type(v, jnp.int32)
    lsb = jnp.bitwise_and(lax.shift_right_logical(b, 16), 1)
    b = jnp.bitwise_and(b + 0x7FFF + lsb, jnp.int32(-65536))
    return lax.bitcast_convert_type(b, jnp.float32)


def _bf16h(v):
    b = lax.bitcast_convert_type(v, jnp.int32)
    b = jnp.bitwise_and(b + 0x8000, jnp.int32(-65536))
    return lax.bitcast_convert_type(b, jnp.float32)


def _rcp16(x):
    r = 1.0 / x
    return r * (2.0 - x * r)


def _exp16(x):
    x = jnp.maximum(x, -87.0)
    z = x * LOG2E
    zi = (z - 0.5).astype(jnp.int32)
    zf = z - zi.astype(jnp.float32)
    t = zf * LN2
    p = 1.0 + t * (1.0 + t * (1.0 / 2 + t * (1.0 / 6 + t * (1.0 / 24 + t * (
        1.0 / 120 + t * (1.0 / 720 + t * (1.0 / 5040)))))))
    scale = lax.bitcast_convert_type(
        lax.shift_left(zi + 127, jnp.int32(23)), jnp.float32)
    return p * scale


def _log16(x):
    bits = lax.bitcast_convert_type(x, jnp.int32)
    e = jnp.bitwise_and(lax.shift_right_logical(bits, 23), 0xFF)
    ef = (e - 127).astype(jnp.float32)
    mbits = jnp.bitwise_or(jnp.bitwise_and(bits, 0x007FFFFF), 0x3F800000)
    m = lax.bitcast_convert_type(mbits, jnp.float32)
    s = (m - 1.0) * _rcp16(m + 1.0)
    s2 = s * s
    p = 2.0 * s * (1.0 + s2 * (1.0 / 3.0 + s2 * (0.2 + s2 * (1.0 / 7.0 + s2 * (1.0 / 9.0)))))
    return ef * LN2 + p


def _make_sc_kernel(n, k, d, nmem):
    info = plsc.get_sparse_core_info()
    nc, ns = info.num_cores, info.num_subcores
    nw = nc * ns
    tpt = n // nw
    nch = d // L
    mesh = plsc.VectorSubcoreMesh(core_axis_name="c", subcore_axis_name="s")

    def body(y_hbm, p_hbm, lam_hbm, m_hbm, kset_hbm,
             pnew_hbm, lamnew_hbm,
             kset_v, p_v, pnew_v, t_v, y_v, lam_v,
             yp_v, xi_v, lamnew_v,
             gsem0, gsem1, iosem0, iosem1, osem0, osem1):
        wid = lax.axis_index("s") * nc + lax.axis_index("c")
        base = wid * tpt
        gsems = (gsem0, gsem1)
        iosems = (iosem0, iosem1)
        osems = (osem0, osem1)

        pltpu.sync_copy(kset_hbm.at[pl.ds(base, tpt)], kset_v)
        pltpu.sync_copy(p_hbm.at[pl.ds(base, tpt)], p_v)

        def fire_in(tl, b):
            tok = base + tl
            pltpu.async_copy(m_hbm.at[kset_v.at[tl]], t_v.at[b], gsems[b])
            pltpu.async_copy(y_hbm.at[tok], y_v.at[b], iosems[b])
            pltpu.async_copy(lam_hbm.at[tok], lam_v.at[b], iosems[b])

        def wait_in(tl, b):
            tok = base + tl
            pltpu.make_async_copy(m_hbm.at[kset_v.at[tl]], t_v.at[b], gsems[b]).wait()
            pltpu.make_async_copy(y_hbm.at[tok], y_v.at[b], iosems[b]).wait()
            pltpu.make_async_copy(lam_hbm.at[tok], lam_v.at[b], iosems[b]).wait()

        def weighted_colsum(b, w0, w1):
            def colsum(c):
                sl = pl.ds(c * L, L)
                acc = jnp.zeros((L,), jnp.float32)
                for j in range(k):
                    w = w0[j] if j < L else w1[j - L]
                    acc = acc + w * _bf16h(t_v[b, j, sl])
                return acc, sl
            return colsum

        def token_compute(t, b):
            p0 = p_v[t, 0:L]
            p1 = p_v[t, L:2 * L]
            psum = jnp.sum(p0) + jnp.sum(p1) + EPS
            inv = _rcp16(jnp.broadcast_to(psum, (L,)))
            pn0 = p0 * inv
            pn1 = p1 * inv

            colsum_pn = weighted_colsum(b, _bf16r(pn0), _bf16r(pn1))

            @plsc.parallel_loop(0, nch)
            def pass1(c):
                acc, sl = colsum_pn(c)
                yp_v[sl] = acc
                xi_v[sl] = _bf16r(lam_v[b, sl] + RHO * (y_v[b, sl] - acc))

            lanes = lax.iota(jnp.int32, L)
            sc0 = sc1 = vv0 = vv1 = q0 = q1 = jnp.zeros((L,), jnp.float32)
            for jb in range(k // 8):
                def chunk(c, accs, jb=jb):
                    sl = pl.ds(c * L, L)
                    xi = xi_v[sl]
                    yp = yp_v[sl]
                    out = []
                    for r in range(8):
                        tv = t_v[b, jb * 8 + r, sl]
                        out.append(accs[3 * r] + _bf16h(tv) * xi)
                        out.append(accs[3 * r + 1] + tv * yp)
                        out.append(accs[3 * r + 2] + tv * tv)
                    return tuple(out)
                accs = plsc.parallel_loop(
                    0, nch,
                    carry=tuple(jnp.zeros((L,), jnp.float32) for _ in range(24)),
                )(chunk)
                for r in range(8):
                    j = jb * 8 + r
                    msk = lanes == (j % L)
                    ssc = jnp.sum(accs[3 * r])
                    svv = jnp.sum(accs[3 * r + 1])
                    sqq = jnp.sum(accs[3 * r + 2])
                    if j < L:
                        sc0 = jnp.where(msk, ssc, sc0)
                        vv0 = jnp.where(msk, svv, vv0)
                        q0 = jnp.where(msk, sqq, q0)
                    else:
                        sc1 = jnp.where(msk, ssc, sc1)
                        vv1 = jnp.where(msk, svv, vv1)
                        q1 = jnp.where(msk, sqq, q1)

            sm0 = q0 - 2.0 * vv0
            sm1 = q1 - 2.0 * vv1
            lg0 = _log16(pn0 + EPS) - BETA * sc0 - TAU * sm0
            lg1 = _log16(pn1 + EPS) - BETA * sc1 - TAU * sm1
            mx = jnp.maximum(jnp.max(lg0), jnp.max(lg1))
            e0 = _exp16(lg0 - mx)
            e1 = _exp16(lg1 - mx)
            iz = _rcp16(jnp.broadcast_to(jnp.sum(e0) + jnp.sum(e1), (L,)))
            pw0 = e0 * iz
            pw1 = e1 * iz
            pnew_v[t, 0:L] = pw0
            pnew_v[t, L:2 * L] = pw1

            colsum_pw = weighted_colsum(b, _bf16r(pw0), _bf16r(pw1))

            @plsc.parallel_loop(0, nch)
            def pass3(c):
                acc, sl = colsum_pw(c)
                lamnew_v[b, sl] = lam_v[b, sl] + RHO * (y_v[b, sl] - acc)

        fire_in(0, 0)

        def pair_body(i, _):
            for tt in range(2):
                b = tt
                tl = 2 * i + tt

                @pl.when(tl + 1 < tpt)
                def _():
                    fire_in(tl + 1, 1 - b)
                wait_in(tl, b)

                @pl.when(tl >= 2)
                def _():
                    pltpu.make_async_copy(
                        lamnew_v.at[b], lamnew_hbm.at[base + tl - 2],
                        osems[b]).wait()
                token_compute(tl, b)
                pltpu.async_copy(lamnew_v.at[b], lamnew_hbm.at[base + tl],
                                 osems[b])
            return 0

        lax.fori_loop(0, tpt // 2, pair_body, 0, unroll=False)
        pltpu.make_async_copy(lamnew_v.at[0], lamnew_hbm.at[base + tpt - 2],
                              osems[0]).wait()
        pltpu.make_async_copy(lamnew_v.at[1], lamnew_hbm.at[base + tpt - 1],
                              osems[1]).wait()
        pltpu.sync_copy(pnew_v, pnew_hbm.at[pl.ds(base, tpt)])

    return pl.kernel(
        body,
        out_type=(jax.ShapeDtypeStruct((n, k), jnp.float32),
                  jax.ShapeDtypeStruct((n, d), jnp.float32)),
        mesh=mesh,
        compiler_params=pltpu.CompilerParams(needs_layout_passes=False),
        scratch_types=[
            pltpu.VMEM((tpt, k), jnp.int32),
            pltpu.VMEM((tpt, k), jnp.float32),
            pltpu.VMEM((tpt, k), jnp.float32),
            pltpu.VMEM((2, k, d), jnp.float32),
            pltpu.VMEM((2, d), jnp.float32),
            pltpu.VMEM((2, d), jnp.float32),
            pltpu.VMEM((d,), jnp.float32),
            pltpu.VMEM((d,), jnp.float32),
            pltpu.VMEM((2, d), jnp.float32),
            pltpu.SemaphoreType.DMA,
            pltpu.SemaphoreType.DMA,
            pltpu.SemaphoreType.DMA,
            pltpu.SemaphoreType.DMA,
            pltpu.SemaphoreType.DMA,
            pltpu.SemaphoreType.DMA,
        ],
    )


def kernel(Y, P, Lam, M, Kset):
    n, d = Y.shape
    k = P.shape[1]
    nmem = M.shape[0]
    sc = _make_sc_kernel(n, k, d, nmem)
    return sc(Y, P, Lam, M, Kset)
